# baseline (speedup 1.0000x reference)
import functools

import jax
import jax.numpy as jnp
from jax import lax
from jax.experimental import pallas as pl
from jax.experimental.pallas import tpu as pltpu
from jax.experimental.pallas import tpu_sc as plsc

INTERP = False

_L = 16


def _dg(vec, idx):
    return lax.gather(
        vec, idx[:, None],
        lax.GatherDimensionNumbers(
            offset_dims=(), collapsed_slice_dims=(0,), start_index_map=(0,)),
        slice_sizes=(1,),
        mode=lax.GatherScatterMode.PROMISE_IN_BOUNDS)


def _router_body(x_ref, w_ref, b_ref, e_ref, g_ref, ps_ref, cnt_ref, z_ref):
    i = pl.program_id(0)
    Ee = w_ref.shape[1]
    logits = jnp.dot(x_ref[...], w_ref[...],
                     preferred_element_type=jnp.float32) + b_ref[...]
    lane = lax.broadcasted_iota(jnp.int32, logits.shape, 1)
    m1 = jnp.max(logits, axis=1, keepdims=True)
    i1 = jnp.min(jnp.where(logits == m1, lane, Ee), axis=1, keepdims=True)
    masked = jnp.where(lane == i1, -jnp.inf, logits)
    m2 = jnp.max(masked, axis=1, keepdims=True)
    i2 = jnp.min(jnp.where(masked == m2, lane, Ee), axis=1, keepdims=True)
    d = jnp.exp(m2 - m1)
    g1 = 1.0 / (1.0 + d)
    g2 = d / (1.0 + d)
    e_ref[...] = jnp.concatenate([i1, i2], axis=1)
    g_ref[...] = jnp.concatenate([g1, g2], axis=1)
    pexp = jnp.exp(logits - m1)
    sexp = jnp.sum(pexp, axis=1, keepdims=True)
    ps = jnp.sum(pexp / sexp, axis=0, keepdims=True)
    lse = m1 + jnp.log(sexp)
    z = jnp.sum(lse * lse, axis=0, keepdims=True)
    oh = (lane == i1).astype(jnp.float32) + (lane == i2).astype(jnp.float32)
    cnt = jnp.sum(oh, axis=0, keepdims=True)

    @pl.when(i == 0)
    def _():
        ps_ref[...] = jnp.zeros_like(ps_ref)
        cnt_ref[...] = jnp.zeros_like(cnt_ref)
        z_ref[...] = jnp.zeros_like(z_ref)

    ps_ref[...] += ps
    cnt_ref[...] += cnt
    z_ref[...] += z


def _router(x_flat, Wr, br, tblk=512):
    T, Dd = x_flat.shape
    Ee = Wr.shape[1]
    outs = (
        jax.ShapeDtypeStruct((T, 2), jnp.int32),
        jax.ShapeDtypeStruct((T, 2), jnp.float32),
        jax.ShapeDtypeStruct((1, Ee), jnp.float32),
        jax.ShapeDtypeStruct((1, Ee), jnp.float32),
        jax.ShapeDtypeStruct((1, 1), jnp.float32),
    )
    return pl.pallas_call(
        _router_body,
        grid=(T // tblk,),
        in_specs=[
            pl.BlockSpec((tblk, Dd), lambda i: (i, 0)),
            pl.BlockSpec((Dd, Ee), lambda i: (0, 0)),
            pl.BlockSpec((1, Ee), lambda i: (0, 0)),
        ],
        out_specs=(
            pl.BlockSpec((tblk, 2), lambda i: (i, 0)),
            pl.BlockSpec((tblk, 2), lambda i: (i, 0)),
            pl.BlockSpec((1, Ee), lambda i: (0, 0)),
            pl.BlockSpec((1, Ee), lambda i: (0, 0)),
            pl.BlockSpec((1, 1), lambda i: (0, 0)),
        ),
        out_shape=outs, interpret=INTERP,
    )(x_flat, Wr, br.reshape(1, Ee))


def _dispatch(eids, gflat, counts16, x_flat, blk, nblk_max):
    TK = eids.shape[0]
    T, Dd = x_flat.shape
    NW = 32
    CH = TK // NW
    NCV = CH // _L
    G = 32
    NG = CH // G
    mesh = plsc.VectorSubcoreMesh(core_axis_name="c", subcore_axis_name="s")

    @functools.partial(
        pl.kernel, mesh=mesh,
        compiler_params=pltpu.CompilerParams(needs_layout_passes=False),
        out_type=[
            jax.ShapeDtypeStruct((TK,), jnp.int32),
            jax.ShapeDtypeStruct((32,), jnp.int32),
            jax.ShapeDtypeStruct((16,), jnp.int32),
            jax.ShapeDtypeStruct((nblk_max * blk, Dd), jnp.float32),
            jax.ShapeDtypeStruct((nblk_max * blk,), jnp.float32),
        ],
        scratch_types=[
            pltpu.VMEM((TK,), jnp.int32),
            pltpu.VMEM((16,), jnp.int32),
            pltpu.VMEM((CH,), jnp.int32),
            pltpu.VMEM((NG, G), jnp.int32),
            pltpu.VMEM((NG, G), jnp.int32),
            pltpu.VMEM((NG, G), jnp.float32),
            pltpu.VMEM((CH,), jnp.float32),
            pltpu.VMEM((NG, G, Dd), jnp.float32),
            pltpu.VMEM((32,), jnp.int32),
            pltpu.VMEM((16,), jnp.int32),
            [pltpu.SemaphoreType.DMA] * 4,
            [pltpu.SemaphoreType.DMA] * 4,
        ],
    )
    def body(eids_hbm, g_hbm, cnt_hbm, x_hbm, pos_hbm, gid_hbm, nused_hbm,
             xpad_hbm, gpad_hbm, eids_v, cnt_v, pos_v, posg_v, tokg_v, gbuf_v,
             gflat_v, rows_v, gid_v, nu_v, gsems, ssems):
        wid = lax.axis_index("s") * 2 + lax.axis_index("c")
        lane = lax.broadcasted_iota(jnp.int32, (_L,), 0)
        my_pair = wid * CH
        for i in range(NCV):
            tokg_v[i // 2, pl.ds((i % 2) * _L, _L)] = \
                (my_pair + i * _L + lane) >> 1
        ghs = [pltpu.async_copy(x_hbm.at[tokg_v.at[j]], rows_v.at[j],
                                gsems[j]) for j in range(NG)]
        pltpu.sync_copy(eids_hbm, eids_v)
        pltpu.sync_copy(cnt_hbm, cnt_v)
        pltpu.sync_copy(g_hbm.at[pl.ds(my_pair, CH)], gflat_v)
        for i in range(NCV):
            gbuf_v[i // 2, pl.ds((i % 2) * _L, _L)] = \
                gflat_v[pl.ds(i * _L, _L)]
        counts = cnt_v[...]

        def scan_body(v, accs):
            ids = eids_v[pl.ds(v * _L, _L)]
            return tuple(a + jnp.where(ids == e, 1, 0)
                         for e, a in enumerate(accs))

        accs = lax.fori_loop(0, wid * NCV, scan_body,
                             tuple(jnp.zeros((_L,), jnp.int32)
                                   for _ in range(8)))
        before = jnp.zeros((_L,), jnp.int32)
        for e in range(8):
            before = jnp.where(lane == e, jnp.sum(accs[e]), before)

        nblk = (counts + (blk - 1)) >> 8
        incl = plsc.cumsum(nblk)
        blk_start = incl - nblk
        padded_start = blk_start * blk
        base = padded_start + before

        for i in range(NCV):
            ids = eids_v[pl.ds(my_pair + i * _L, _L)]
            baseg = _dg(base, ids)
            rank = jnp.zeros((_L,), jnp.int32)
            add = jnp.zeros((_L,), jnp.int32)
            for e in range(8):
                m = ids == e
                mi = jnp.where(m, 1, 0)
                cs = plsc.cumsum(mi)
                rank = jnp.where(m, cs - 1, rank)
                add = jnp.where(lane == e, cs[_L - 1], add)
            pos = baseg + rank
            base = base + add
            pos_v[pl.ds(i * _L, _L)] = pos
            posg_v[i // 2, pl.ds((i % 2) * _L, _L)] = pos

        pltpu.sync_copy(pos_v, pos_hbm.at[pl.ds(my_pair, CH)])
        shs = []
        for j in range(NG):
            ghs[j].wait()
            shs.append(pltpu.async_copy(rows_v.at[j],
                                        xpad_hbm.at[posg_v.at[j]], ssems[j]))
            shs.append(pltpu.async_copy(gbuf_v.at[j],
                                        gpad_hbm.at[posg_v.at[j]], gsems[j]))

        @pl.when(wid == 0)
        def _():
            nused = jnp.sum(nblk)
            lastg = jnp.max(jnp.where(nblk > 0, lane, -1))
            for r in range(2):
                bv = lane + r * _L
                gv = jnp.zeros((_L,), jnp.int32)
                for e in range(8):
                    st_e = _dg(blk_start, jnp.full((_L,), e, jnp.int32))
                    gv = gv + jnp.where(st_e <= bv, 1, 0)
                gv = jnp.minimum(gv - 1, lastg)
                gid_v[pl.ds(r * _L, _L)] = gv
            nu_v[...] = jnp.where(lane == 0, nused, 0)
            pltpu.sync_copy(gid_v, gid_hbm)
            pltpu.sync_copy(nu_v, nused_hbm)

        for h in shs:
            h.wait()

    return body(eids, gflat, counts16, x_flat)


def _combine(y_pad, pos, T, Dd):
    TK = pos.shape[0]
    NW = 32
    PW = TK // NW
    TW = PW // 2
    G = 32
    NG = PW // G
    TG = G // 2
    NSL = Dd // _L
    mesh = plsc.VectorSubcoreMesh(core_axis_name="c", subcore_axis_name="s")

    @functools.partial(
        pl.kernel, mesh=mesh,
        compiler_params=pltpu.CompilerParams(needs_layout_passes=False),
        out_type=jax.ShapeDtypeStruct((T, Dd), jnp.float32),
        scratch_types=[
            pltpu.VMEM((PW,), jnp.int32),
            pltpu.VMEM((NG, G, Dd), jnp.float32),
            pltpu.VMEM((2, TG, Dd), jnp.float32),
            [pltpu.SemaphoreType.DMA] * 4,
            [pltpu.SemaphoreType.DMA] * 2,
        ],
    )
    def body(y_hbm, pos_hbm, out_hbm, pos_v, rows_v, out_v, gsems, osems):
        wid = lax.axis_index("s") * 2 + lax.axis_index("c")
        my_pair = wid * PW
        my_tok = wid * TW
        pltpu.sync_copy(pos_hbm.at[pl.ds(my_pair, PW)], pos_v)
        ghs = [pltpu.async_copy(
            y_hbm.at[pos_v.at[pl.ds(j * G, G)]], rows_v.at[j], gsems[j])
            for j in range(NG)]
        ohs = [None] * NG
        for j in range(NG):
            b = j % 2
            ghs[j].wait()
            if j >= 2:
                ohs[j - 2].wait()

            def tok_body(t, _):
                for sl0 in range(0, NSL, 8):
                    r0s = [rows_v[j, 2 * t, pl.ds((sl0 + k) * _L, _L)]
                           for k in range(8)]
                    r1s = [rows_v[j, 2 * t + 1, pl.ds((sl0 + k) * _L, _L)]
                           for k in range(8)]
                    outs = [a + c for a, c in zip(r0s, r1s)]
                    for k in range(8):
                        out_v[b, t, pl.ds((sl0 + k) * _L, _L)] = outs[k]
                return 0

            lax.fori_loop(0, TG, tok_body, 0)
            ohs[j] = pltpu.async_copy(
                out_v.at[b], out_hbm.at[pl.ds(my_tok + j * TG, TG)], osems[b])
        ohs[NG - 2].wait()
        ohs[NG - 1].wait()

    return body(y_pad, pos)


def _ffn_body(gid_ref, nu_ref, x_ref, g_ref, w1_ref, b1_ref, w3_ref, b3_ref,
              w2_ref, b2_ref, y_ref):
    i = pl.program_id(0)

    @pl.when(i < nu_ref[0])
    def _():
        xb = x_ref[...]
        h1 = jnp.dot(xb, w1_ref[0], preferred_element_type=jnp.float32) \
            + b1_ref[0]
        h3 = jnp.dot(xb, w3_ref[0], preferred_element_type=jnp.float32) \
            + b3_ref[0]
        h = h1 * lax.logistic(h1) * h3
        y = jnp.dot(h, w2_ref[0], preferred_element_type=jnp.float32) \
            + b2_ref[0]
        y_ref[...] = y * g_ref[...]


def _expert_ffn(x_pad, g_pad, W1, b1, W3, b3, W2, b2, gid, nused, blk, nblk):
    Ee, Dd, Ff = W1.shape
    b1r = b1.reshape(Ee, 1, Ff)
    b3r = b3.reshape(Ee, 1, Ff)
    b2r = b2.reshape(Ee, 1, Dd)
    grid_spec = pltpu.PrefetchScalarGridSpec(
        num_scalar_prefetch=2,
        grid=(nblk,),
        in_specs=[
            pl.BlockSpec((blk, Dd),
                         lambda i, g, n: (jnp.minimum(i, n[0] - 1), 0)),
            pl.BlockSpec((blk, 1),
                         lambda i, g, n: (jnp.minimum(i, n[0] - 1), 0)),
            pl.BlockSpec((1, Dd, Ff), lambda i, g, n: (g[i], 0, 0)),
            pl.BlockSpec((1, 1, Ff), lambda i, g, n: (g[i], 0, 0)),
            pl.BlockSpec((1, Dd, Ff), lambda i, g, n: (g[i], 0, 0)),
            pl.BlockSpec((1, 1, Ff), lambda i, g, n: (g[i], 0, 0)),
            pl.BlockSpec((1, Ff, Dd), lambda i, g, n: (g[i], 0, 0)),
            pl.BlockSpec((1, 1, Dd), lambda i, g, n: (g[i], 0, 0)),
        ],
        out_specs=pl.BlockSpec(
            (blk, Dd), lambda i, g, n: (jnp.where(i < n[0], i, nblk), 0)),
    )
    return pl.pallas_call(
        _ffn_body, grid_spec=grid_spec,
        out_shape=jax.ShapeDtypeStruct(((nblk + 1) * blk, Dd), jnp.float32),
        interpret=INTERP,
    )(gid, nused, x_pad, g_pad, W1, b1r, W3, b3r, W2, b2r)


def _shared_body(x_ref, w1_ref, b1_ref, w3_ref, b3_ref, w2_ref, b2_ref,
                 o_ref):
    xb = x_ref[...]
    h1 = jnp.dot(xb, w1_ref[...], preferred_element_type=jnp.float32) \
        + b1_ref[...]
    h3 = jnp.dot(xb, w3_ref[...], preferred_element_type=jnp.float32) \
        + b3_ref[...]
    h = h1 * lax.logistic(h1) * h3
    o_ref[0] = jnp.dot(h, w2_ref[...], preferred_element_type=jnp.float32) \
        + b2_ref[...]


def _shared_ffn(xin, SW1, Sb1, SW3, Sb3, SW2, Sb2, Bb, Ss, tblk=256):
    T, Dd = xin.shape
    Ff = SW1.shape[1]
    return pl.pallas_call(
        _shared_body,
        grid=(T // tblk,),
        in_specs=[
            pl.BlockSpec((tblk, Dd), lambda i: (i, 0)),
            pl.BlockSpec((Dd, Ff), lambda i: (0, 0)),
            pl.BlockSpec((1, Ff), lambda i: (0, 0)),
            pl.BlockSpec((Dd, Ff), lambda i: (0, 0)),
            pl.BlockSpec((1, Ff), lambda i: (0, 0)),
            pl.BlockSpec((Ff, Dd), lambda i: (0, 0)),
            pl.BlockSpec((1, Dd), lambda i: (0, 0)),
        ],
        out_specs=pl.BlockSpec((1, tblk, Dd), lambda i: (0, i, 0)),
        out_shape=jax.ShapeDtypeStruct((Bb, Ss, Dd), jnp.float32),
        interpret=INTERP,
    )(xin, SW1, Sb1.reshape(1, Ff), SW3, Sb3.reshape(1, Ff), SW2,
      Sb2.reshape(1, Dd))


def kernel(x, Wr, br, W1, b1, W2, b2, W3, b3, SW1, Sb1, SW2, Sb2, SW3, Sb3):
    Bb, Ss, Dd = x.shape
    T = Bb * Ss
    Ee = Wr.shape[1]
    Kk = 2
    BLK = 256
    NBLK = T * Kk // BLK + Ee
    x_flat = x.reshape(T, Dd)

    eids2, gates2, psum, cnt, zsum = _router(x_flat, Wr, br)
    counts_f = cnt[0, :Ee]
    z_loss = zsum[0, 0] / T
    p_mean = psum[0, :Ee] / T
    f_frac = counts_f / (T * Kk)
    lb_loss = Ee * jnp.sum(p_mean * f_frac)

    counts16 = jnp.zeros((16,), jnp.int32).at[:Ee].set(
        counts_f.astype(jnp.int32))
    eflat = eids2.reshape(T * Kk)
    gflat = gates2.reshape(T * Kk)

    pos, gid, nused, x_pad, g_pad = _dispatch(eflat, gflat, counts16,
                                              x_flat, BLK, NBLK)
    y_pad = _expert_ffn(x_pad, g_pad.reshape(NBLK * BLK, 1), W1, b1, W3, b3,
                        W2, b2, gid, nused, BLK, NBLK)
    comb = _combine(y_pad, pos, T, Dd)
    out = _shared_ffn(comb, SW1, Sb1, SW3, Sb3, SW2, Sb2, Bb, Ss)
    return (out, f_frac, z_loss, z_loss * 0.001,
            lb_loss, lb_loss * 0.1)


# speedup vs baseline: 1.1719x; 1.1719x over previous
import functools

import jax
import jax.numpy as jnp
from jax import lax
from jax.experimental import pallas as pl
from jax.experimental.pallas import tpu as pltpu
from jax.experimental.pallas import tpu_sc as plsc

INTERP = False

_L = 16


def _dg(vec, idx):
    return lax.gather(
        vec, idx[:, None],
        lax.GatherDimensionNumbers(
            offset_dims=(), collapsed_slice_dims=(0,), start_index_map=(0,)),
        slice_sizes=(1,),
        mode=lax.GatherScatterMode.PROMISE_IN_BOUNDS)


def _router_body(x_ref, w_ref, b_ref, e_ref, g_ref, ps_ref, cnt_ref, z_ref):
    i = pl.program_id(0)
    Ee = w_ref.shape[1]
    logits = jnp.dot(x_ref[...], w_ref[...],
                     preferred_element_type=jnp.float32) + b_ref[...]
    lane = lax.broadcasted_iota(jnp.int32, logits.shape, 1)
    m1 = jnp.max(logits, axis=1, keepdims=True)
    i1 = jnp.min(jnp.where(logits == m1, lane, Ee), axis=1, keepdims=True)
    masked = jnp.where(lane == i1, -jnp.inf, logits)
    m2 = jnp.max(masked, axis=1, keepdims=True)
    i2 = jnp.min(jnp.where(masked == m2, lane, Ee), axis=1, keepdims=True)
    d = jnp.exp(m2 - m1)
    g1 = 1.0 / (1.0 + d)
    g2 = d / (1.0 + d)
    e_ref[...] = jnp.concatenate([i1, i2], axis=1)
    g_ref[...] = jnp.concatenate([g1, g2], axis=1)
    pexp = jnp.exp(logits - m1)
    sexp = jnp.sum(pexp, axis=1, keepdims=True)
    ps = jnp.sum(pexp / sexp, axis=0, keepdims=True)
    lse = m1 + jnp.log(sexp)
    z = jnp.sum(lse * lse, axis=0, keepdims=True)
    oh = (lane == i1).astype(jnp.float32) + (lane == i2).astype(jnp.float32)
    cnt = jnp.sum(oh, axis=0, keepdims=True)

    @pl.when(i == 0)
    def _():
        ps_ref[...] = jnp.zeros_like(ps_ref)
        cnt_ref[...] = jnp.zeros_like(cnt_ref)
        z_ref[...] = jnp.zeros_like(z_ref)

    ps_ref[...] += ps
    cnt_ref[...] += cnt
    z_ref[...] += z


def _router(x_flat, Wr, br, tblk=512):
    T, Dd = x_flat.shape
    Ee = Wr.shape[1]
    outs = (
        jax.ShapeDtypeStruct((T, 2), jnp.int32),
        jax.ShapeDtypeStruct((T, 2), jnp.float32),
        jax.ShapeDtypeStruct((1, Ee), jnp.float32),
        jax.ShapeDtypeStruct((1, Ee), jnp.float32),
        jax.ShapeDtypeStruct((1, 1), jnp.float32),
    )
    return pl.pallas_call(
        _router_body,
        grid=(T // tblk,),
        in_specs=[
            pl.BlockSpec((tblk, Dd), lambda i: (i, 0)),
            pl.BlockSpec((Dd, Ee), lambda i: (0, 0)),
            pl.BlockSpec((1, Ee), lambda i: (0, 0)),
        ],
        out_specs=(
            pl.BlockSpec((tblk, 2), lambda i: (i, 0)),
            pl.BlockSpec((tblk, 2), lambda i: (i, 0)),
            pl.BlockSpec((1, Ee), lambda i: (0, 0)),
            pl.BlockSpec((1, Ee), lambda i: (0, 0)),
            pl.BlockSpec((1, 1), lambda i: (0, 0)),
        ),
        out_shape=outs, interpret=INTERP,
    )(x_flat, Wr, br.reshape(1, Ee))


def _dispatch(eids, counts16, x_flat, blk, nblk_max):
    TK = eids.shape[0]
    T, Dd = x_flat.shape
    NW = 32
    CH = TK // NW
    NCV = CH // _L
    G = 32
    NG = CH // G
    mesh = plsc.VectorSubcoreMesh(core_axis_name="c", subcore_axis_name="s")

    @functools.partial(
        pl.kernel, mesh=mesh,
        compiler_params=pltpu.CompilerParams(needs_layout_passes=False),
        out_type=[
            jax.ShapeDtypeStruct((TK,), jnp.int32),
            jax.ShapeDtypeStruct((32,), jnp.int32),
            jax.ShapeDtypeStruct((16,), jnp.int32),
            jax.ShapeDtypeStruct((nblk_max * blk, Dd), jnp.float32),
        ],
        scratch_types=[
            pltpu.VMEM((TK,), jnp.int32),
            pltpu.VMEM((16,), jnp.int32),
            pltpu.VMEM((CH,), jnp.int32),
            pltpu.VMEM((NG, G), jnp.int32),
            pltpu.VMEM((NG, G), jnp.int32),
            pltpu.VMEM((NG, G, Dd), jnp.float32),
            pltpu.VMEM((32,), jnp.int32),
            pltpu.VMEM((16,), jnp.int32),
            [pltpu.SemaphoreType.DMA] * 4,
            [pltpu.SemaphoreType.DMA] * 4,
        ],
    )
    def body(eids_hbm, cnt_hbm, x_hbm, pos_hbm, gid_hbm, nused_hbm,
             xpad_hbm, eids_v, cnt_v, pos_v, posg_v, tokg_v,
             rows_v, gid_v, nu_v, gsems, ssems):
        wid = lax.axis_index("s") * 2 + lax.axis_index("c")
        lane = lax.broadcasted_iota(jnp.int32, (_L,), 0)
        my_pair = wid * CH
        for i in range(NCV):
            tokg_v[i // 2, pl.ds((i % 2) * _L, _L)] = \
                (my_pair + i * _L + lane) >> 1
        ghs = [pltpu.async_copy(x_hbm.at[tokg_v.at[j]], rows_v.at[j],
                                gsems[j]) for j in range(NG)]
        pltpu.sync_copy(eids_hbm, eids_v)
        pltpu.sync_copy(cnt_hbm, cnt_v)
        counts = cnt_v[...]

        def scan_body(v, accs):
            ids = eids_v[pl.ds(v * _L, _L)]
            return tuple(a + jnp.where(ids == e, 1, 0)
                         for e, a in enumerate(accs))

        accs = lax.fori_loop(0, wid * NCV, scan_body,
                             tuple(jnp.zeros((_L,), jnp.int32)
                                   for _ in range(8)))
        before = jnp.zeros((_L,), jnp.int32)
        for e in range(8):
            before = jnp.where(lane == e, jnp.sum(accs[e]), before)

        nblk = (counts + (blk - 1)) >> 8
        incl = plsc.cumsum(nblk)
        blk_start = incl - nblk
        padded_start = blk_start * blk
        base = padded_start + before

        for i in range(NCV):
            ids = eids_v[pl.ds(my_pair + i * _L, _L)]
            baseg = _dg(base, ids)
            rank = jnp.zeros((_L,), jnp.int32)
            add = jnp.zeros((_L,), jnp.int32)
            for e in range(8):
                m = ids == e
                mi = jnp.where(m, 1, 0)
                cs = plsc.cumsum(mi)
                rank = jnp.where(m, cs - 1, rank)
                add = jnp.where(lane == e, cs[_L - 1], add)
            pos = baseg + rank
            base = base + add
            pos_v[pl.ds(i * _L, _L)] = pos
            posg_v[i // 2, pl.ds((i % 2) * _L, _L)] = pos

        pltpu.sync_copy(pos_v, pos_hbm.at[pl.ds(my_pair, CH)])
        shs = []
        for j in range(NG):
            ghs[j].wait()
            shs.append(pltpu.async_copy(rows_v.at[j],
                                        xpad_hbm.at[posg_v.at[j]], ssems[j]))

        @pl.when(wid == 0)
        def _():
            nused = jnp.sum(nblk)
            lastg = jnp.max(jnp.where(nblk > 0, lane, -1))
            for r in range(2):
                bv = lane + r * _L
                gv = jnp.zeros((_L,), jnp.int32)
                for e in range(8):
                    st_e = _dg(blk_start, jnp.full((_L,), e, jnp.int32))
                    gv = gv + jnp.where(st_e <= bv, 1, 0)
                gv = jnp.minimum(gv - 1, lastg)
                gid_v[pl.ds(r * _L, _L)] = gv
            nu_v[...] = jnp.where(lane == 0, nused, 0)
            pltpu.sync_copy(gid_v, gid_hbm)
            pltpu.sync_copy(nu_v, nused_hbm)

        for h in shs:
            h.wait()

    return body(eids, counts16, x_flat)


def _combine(y_pad, pos, gates_flat, T, Dd):
    TK = pos.shape[0]
    NW = 32
    PW = TK // NW
    TW = PW // 2
    G = 32
    NG = PW // G
    TG = G // 2
    NSL = Dd // _L
    mesh = plsc.VectorSubcoreMesh(core_axis_name="c", subcore_axis_name="s")

    @functools.partial(
        pl.kernel, mesh=mesh,
        compiler_params=pltpu.CompilerParams(needs_layout_passes=False),
        out_type=jax.ShapeDtypeStruct((T, Dd), jnp.float32),
        scratch_types=[
            pltpu.VMEM((PW,), jnp.int32),
            pltpu.VMEM((PW + _L,), jnp.float32),
            pltpu.VMEM((NG, G, Dd), jnp.float32),
            pltpu.VMEM((2, TG, Dd), jnp.float32),
            [pltpu.SemaphoreType.DMA] * 4,
            [pltpu.SemaphoreType.DMA] * 2,
        ],
    )
    def body(y_hbm, pos_hbm, g_hbm, out_hbm, pos_v, g_v, rows_v, out_v,
             gsems, osems):
        wid = lax.axis_index("s") * 2 + lax.axis_index("c")
        my_pair = wid * PW
        my_tok = wid * TW
        pltpu.sync_copy(pos_hbm.at[pl.ds(my_pair, PW)], pos_v)
        ghs = [pltpu.async_copy(
            y_hbm.at[pos_v.at[pl.ds(j * G, G)]], rows_v.at[j], gsems[j])
            for j in range(NG)]
        pltpu.sync_copy(g_hbm.at[pl.ds(my_pair, PW)], g_v.at[pl.ds(0, PW)])
        ohs = [None] * NG
        for j in range(NG):
            b = j % 2
            ghs[j].wait()
            if j >= 2:
                ohs[j - 2].wait()

            def tok_body(t, _):
                gv = g_v[pl.ds(j * G + 2 * t, _L)]
                g0 = gv[0]
                g1 = gv[1]
                for sl0 in range(0, NSL, 8):
                    r0s = [rows_v[j, 2 * t, pl.ds((sl0 + k) * _L, _L)]
                           for k in range(8)]
                    r1s = [rows_v[j, 2 * t + 1, pl.ds((sl0 + k) * _L, _L)]
                           for k in range(8)]
                    outs = [g0 * a + g1 * c for a, c in zip(r0s, r1s)]
                    for k in range(8):
                        out_v[b, t, pl.ds((sl0 + k) * _L, _L)] = outs[k]
                return 0

            lax.fori_loop(0, TG, tok_body, 0)
            ohs[j] = pltpu.async_copy(
                out_v.at[b], out_hbm.at[pl.ds(my_tok + j * TG, TG)], osems[b])
        ohs[NG - 2].wait()
        ohs[NG - 1].wait()

    return body(y_pad, pos, gates_flat)


def _ffn_body(gid_ref, nu_ref, x_ref, w1_ref, b1_ref, w3_ref, b3_ref,
              w2_ref, b2_ref, y_ref):
    i = pl.program_id(0)

    @pl.when(i < nu_ref[0])
    def _():
        xb = x_ref[...]
        h1 = jnp.dot(xb, w1_ref[0], preferred_element_type=jnp.float32) \
            + b1_ref[0]
        h3 = jnp.dot(xb, w3_ref[0], preferred_element_type=jnp.float32) \
            + b3_ref[0]
        h = h1 * lax.logistic(h1) * h3
        y_ref[...] = jnp.dot(h, w2_ref[0], preferred_element_type=jnp.float32) \
            + b2_ref[0]


def _expert_ffn(x_pad, W1, b1, W3, b3, W2, b2, gid, nused, blk, nblk):
    Ee, Dd, Ff = W1.shape
    b1r = b1.reshape(Ee, 1, Ff)
    b3r = b3.reshape(Ee, 1, Ff)
    b2r = b2.reshape(Ee, 1, Dd)
    grid_spec = pltpu.PrefetchScalarGridSpec(
        num_scalar_prefetch=2,
        grid=(nblk,),
        in_specs=[
            pl.BlockSpec((blk, Dd),
                         lambda i, g, n: (jnp.minimum(i, n[0] - 1), 0)),
            pl.BlockSpec((1, Dd, Ff), lambda i, g, n: (g[i], 0, 0)),
            pl.BlockSpec((1, 1, Ff), lambda i, g, n: (g[i], 0, 0)),
            pl.BlockSpec((1, Dd, Ff), lambda i, g, n: (g[i], 0, 0)),
            pl.BlockSpec((1, 1, Ff), lambda i, g, n: (g[i], 0, 0)),
            pl.BlockSpec((1, Ff, Dd), lambda i, g, n: (g[i], 0, 0)),
            pl.BlockSpec((1, 1, Dd), lambda i, g, n: (g[i], 0, 0)),
        ],
        out_specs=pl.BlockSpec(
            (blk, Dd), lambda i, g, n: (jnp.where(i < n[0], i, nblk), 0)),
    )
    return pl.pallas_call(
        _ffn_body, grid_spec=grid_spec,
        out_shape=jax.ShapeDtypeStruct(((nblk + 1) * blk, Dd), jnp.float32),
        interpret=INTERP,
    )(gid, nused, x_pad, W1, b1r, W3, b3r, W2, b2r)


def _shared_body(x_ref, w1_ref, b1_ref, w3_ref, b3_ref, w2_ref, b2_ref,
                 o_ref):
    xb = x_ref[...]
    h1 = jnp.dot(xb, w1_ref[...], preferred_element_type=jnp.float32) \
        + b1_ref[...]
    h3 = jnp.dot(xb, w3_ref[...], preferred_element_type=jnp.float32) \
        + b3_ref[...]
    h = h1 * lax.logistic(h1) * h3
    o_ref[0] = jnp.dot(h, w2_ref[...], preferred_element_type=jnp.float32) \
        + b2_ref[...]


def _shared_ffn(xin, SW1, Sb1, SW3, Sb3, SW2, Sb2, Bb, Ss, tblk=256):
    T, Dd = xin.shape
    Ff = SW1.shape[1]
    return pl.pallas_call(
        _shared_body,
        grid=(T // tblk,),
        in_specs=[
            pl.BlockSpec((tblk, Dd), lambda i: (i, 0)),
            pl.BlockSpec((Dd, Ff), lambda i: (0, 0)),
            pl.BlockSpec((1, Ff), lambda i: (0, 0)),
            pl.BlockSpec((Dd, Ff), lambda i: (0, 0)),
            pl.BlockSpec((1, Ff), lambda i: (0, 0)),
            pl.BlockSpec((Ff, Dd), lambda i: (0, 0)),
            pl.BlockSpec((1, Dd), lambda i: (0, 0)),
        ],
        out_specs=pl.BlockSpec((1, tblk, Dd), lambda i: (0, i, 0)),
        out_shape=jax.ShapeDtypeStruct((Bb, Ss, Dd), jnp.float32),
        interpret=INTERP,
    )(xin, SW1, Sb1.reshape(1, Ff), SW3, Sb3.reshape(1, Ff), SW2,
      Sb2.reshape(1, Dd))


def kernel(x, Wr, br, W1, b1, W2, b2, W3, b3, SW1, Sb1, SW2, Sb2, SW3, Sb3):
    Bb, Ss, Dd = x.shape
    T = Bb * Ss
    Ee = Wr.shape[1]
    Kk = 2
    BLK = 256
    NBLK = T * Kk // BLK + Ee
    x_flat = x.reshape(T, Dd)

    eids2, gates2, psum, cnt, zsum = _router(x_flat, Wr, br)
    counts_f = cnt[0, :Ee]
    z_loss = zsum[0, 0] / T
    p_mean = psum[0, :Ee] / T
    f_frac = counts_f / (T * Kk)
    lb_loss = Ee * jnp.sum(p_mean * f_frac)

    counts16 = jnp.zeros((16,), jnp.int32).at[:Ee].set(
        counts_f.astype(jnp.int32))
    eflat = eids2.reshape(T * Kk)
    gflat = gates2.reshape(T * Kk)

    pos, gid, nused, x_pad = _dispatch(eflat, counts16, x_flat, BLK, NBLK)
    y_pad = _expert_ffn(x_pad, W1, b1, W3, b3, W2, b2, gid, nused, BLK, NBLK)
    comb = _combine(y_pad, pos, gflat, T, Dd)
    out = _shared_ffn(comb, SW1, Sb1, SW3, Sb3, SW2, Sb2, Bb, Ss)
    return (out, f_frac, z_loss, z_loss * 0.001,
            lb_loss, lb_loss * 0.1)
